# baseline (speedup 1.0000x reference)
import numpy as np
import jax
import jax.numpy as jnp
from jax.experimental import pallas as pl
from jax.experimental.pallas import tpu as pltpu


def _sel1():
    R = np.zeros((5, 2, 4, 2, 2), np.float32)
    for u in range(2):
        for r in range(4):
            for r2 in range(2):
                for hp in range(2):
                    di = 4 * u + r - 2 * r2 - hp
                    if 0 <= di < 5:
                        R[di, u, r, r2, hp] = 1.0
    C = np.zeros((5, 32, 2, 16), np.float32)
    for w_in in range(32):
        for wp in range(2):
            for w2 in range(14):
                dj = w_in - 2 * w2 - wp
                if 0 <= dj < 5:
                    C[dj, w_in, wp, w2] = 1.0
    return R, C


def _sel2():
    R = np.zeros((5, 3, 2, 2), np.float32)
    for t in range(3):
        for r2 in range(2):
            for hp in range(2):
                di = 2 * t + r2 - hp
                if 0 <= di < 5:
                    R[di, t, r2, hp] = 1.0
    C = np.zeros((5, 16, 2, 8), np.float32)
    for w in range(14):
        for wp in range(2):
            for w2 in range(5):
                dj = w - 2 * w2 - wp
                if 0 <= dj < 5:
                    C[dj, w, wp, w2] = 1.0
    return R, C


_R1, _C1 = _sel1()
_R2, _C2 = _sel2()
_BM1 = np.repeat((np.arange(32) % 16 < 14).astype(np.float32), 8)[None, :]
_BM2 = np.repeat((np.arange(8) < 5).astype(np.float32), 16)[None, :]


def _lenet_kernel(x_ref, wq1_ref, b1_ref, wq2_ref, b2_ref,
                  w3_ref, b3_ref, w4_ref, b4_ref, w5_ref, b5_ref,
                  o_ref, a1s_ref):
    j = pl.program_id(1)
    B = x_ref.shape[0]

    pieces = [x_ref[:, ci, pl.ds((j + u) * 128, 128)]
              for u in range(2) for ci in range(3)]
    lhs = jnp.concatenate(pieces, axis=1).astype(jnp.bfloat16)
    y = jnp.dot(lhs, wq1_ref[...], preferred_element_type=jnp.float32)
    h0 = jnp.maximum(jnp.maximum(y[:, 0:128], y[:, 128:256]),
                     jnp.maximum(y[:, 256:384], y[:, 384:512]))
    h1 = jnp.maximum(jnp.maximum(y[:, 512:640], y[:, 640:768]),
                     jnp.maximum(y[:, 768:896], y[:, 896:1024]))
    a1 = jnp.maximum(jnp.concatenate([h0, h1], axis=1) + b1_ref[...], 0.0)
    a1s_ref[j] = a1.astype(jnp.bfloat16)

    @pl.when(j == 6)
    def _tail():
        lhs2 = jnp.concatenate([a1s_ref[t:t + 5] for t in range(3)],
                               axis=2).reshape(5 * B, 768)
        y2 = jnp.dot(lhs2, wq2_ref[...], preferred_element_type=jnp.float32)
        a2 = jnp.maximum(jnp.maximum(y2[:, 0:128], y2[:, 128:256]),
                         jnp.maximum(y2[:, 256:384], y2[:, 384:512]))
        a2 = jnp.maximum(a2 + b2_ref[...], 0.0).reshape(5, B, 128)

        f_in = jnp.concatenate([a2[h] for h in range(5)], axis=1)
        f1 = jnp.maximum(jnp.dot(f_in, w3_ref[...],
                                 preferred_element_type=jnp.float32)
                         + b3_ref[...], 0.0)
        f2 = jnp.maximum(jnp.dot(f1, w4_ref[...],
                                 preferred_element_type=jnp.float32)
                         + b4_ref[...], 0.0)
        logits = jnp.dot(f2, w5_ref[...],
                         preferred_element_type=jnp.float32) + b5_ref[...]
        o_ref[...] = logits[:, :100]


def kernel(x, w1, b1, w2, b2, w3, b3, w4, b4, w5, b5):
    n = x.shape[0]
    bc = n // 2
    xs = x.reshape(n, 3, 1024)

    w1t = w1[:, :8].reshape(5, 5, 3, 8)
    t1 = jnp.tensordot(w1t, _R1, axes=[[0], [0]])
    t1 = jnp.tensordot(t1, _C1, axes=[[0], [0]])
    wq1 = t1.transpose(2, 0, 3, 6, 4, 5, 7, 8, 1).reshape(768, 1024)
    w2t = jnp.pad(w2[:, :16].reshape(5, 5, 6, 16),
                  ((0, 0), (0, 0), (0, 2), (0, 0)))
    t2 = jnp.tensordot(w2t, _R2, axes=[[0], [0]])
    t2 = jnp.tensordot(t2, _C2, axes=[[0], [0]])
    wq2 = t2.transpose(2, 3, 5, 0, 4, 6, 7, 1).reshape(768, 512)
    w3c = jnp.pad(w3.reshape(5, 80, 128),
                  ((0, 0), (0, 48), (0, 0))).reshape(640, 128)
    b1L = jnp.tile(b1[:, :8], (1, 32)) * _BM1
    b2L = jnp.tile(b2[:, :16], (1, 8)) * _BM2

    c2 = lambda i, j: (0, 0)
    out = pl.pallas_call(
        _lenet_kernel,
        out_shape=jax.ShapeDtypeStruct((n, 100), jnp.float32),
        grid=(2, 7),
        in_specs=[
            pl.BlockSpec((bc, 3, 1024), lambda i, j: (i, 0, 0)),
            pl.BlockSpec((768, 1024), c2),
            pl.BlockSpec((1, 256), c2),
            pl.BlockSpec((768, 512), c2),
            pl.BlockSpec((1, 128), c2),
            pl.BlockSpec((640, 128), c2),
            pl.BlockSpec((1, 128), c2),
            pl.BlockSpec((128, 128), c2),
            pl.BlockSpec((1, 128), c2),
            pl.BlockSpec((128, 128), c2),
            pl.BlockSpec((1, 128), c2),
        ],
        out_specs=pl.BlockSpec((bc, 100), lambda i, j: (i, 0)),
        scratch_shapes=[pltpu.VMEM((7, bc, 256), jnp.bfloat16)],
        compiler_params=pltpu.CompilerParams(
            dimension_semantics=("parallel", "arbitrary"),
            vmem_limit_bytes=64 * 1024 * 1024,
        ),
    )(xs, wq1.astype(jnp.bfloat16), b1L, wq2.astype(jnp.bfloat16), b2L,
      w3c, b3, w4, b4, w5, b5)

    return out
